# baseline (speedup 1.0000x reference)
import functools

import jax
import jax.numpy as jnp
from jax import lax
from jax.experimental import pallas as pl
from jax.experimental.pallas import tpu as pltpu
from jax.experimental.pallas import tpu_sc as plsc

NUM_EMBEDDINGS = 100000
EMBEDDING_DIM = 32
NUM_CODEBOOK = 16
NUM_CODEWORD = 32
KD = NUM_CODEBOOK * NUM_CODEWORD

PACK = 4
PACK_BLOCK = 200


def _softmax_compose(c, cb):
    w = NUM_CODEWORD
    m = jnp.max(c, axis=1, keepdims=True)
    e = jnp.exp(c - m)
    s4 = e[:, 0:128] + e[:, 128:256] + e[:, 256:384] + e[:, 384:512]
    s = (s4[:, 0:w] + s4[:, w:2 * w] + s4[:, 2 * w:3 * w]
         + s4[:, 3 * w:4 * w])
    rs = 1.0 / s
    rs128 = jnp.concatenate([rs, rs, rs, rs], axis=1)
    rsfull = jnp.concatenate([rs128] * 4, axis=1)
    p = e * rsfull
    return jnp.dot(p, cb, preferred_element_type=jnp.float32)


def _transform_body(code_ref, cb_ref, out_ref):
    cb = cb_ref[...]
    accs = [
        _softmax_compose(code_ref[:, q * KD:(q + 1) * KD], cb)
        for q in range(PACK)
    ]
    out_ref[...] = jnp.concatenate(accs, axis=1)


def _build_table(code4, cb2):
    rows = NUM_EMBEDDINGS // PACK
    grid = rows // PACK_BLOCK
    return pl.pallas_call(
        _transform_body,
        grid=(grid,),
        in_specs=[
            pl.BlockSpec((PACK_BLOCK, PACK * KD), lambda i: (i, 0)),
            pl.BlockSpec((KD, EMBEDDING_DIM), lambda i: (0, 0)),
        ],
        out_specs=pl.BlockSpec((PACK_BLOCK, PACK * EMBEDDING_DIM),
                               lambda i: (i, 0)),
        out_shape=jax.ShapeDtypeStruct((rows, PACK * EMBEDDING_DIM),
                                       jnp.float32),
    )(code4, cb2)


GATHER_CHUNK = 128


def _gather_rows(table, idx2, n_total, n_chunks_per_worker, nw):
    chunks = n_chunks_per_worker
    mesh = plsc.VectorSubcoreMesh(core_axis_name="c", subcore_axis_name="s")

    @functools.partial(
        pl.kernel,
        mesh=mesh,
        out_type=jax.ShapeDtypeStruct((n_total, EMBEDDING_DIM), jnp.float32),
        compiler_params=pltpu.CompilerParams(use_tc_tiling_on_sc=False),
        scratch_types=[
            pltpu.VMEM((chunks, GATHER_CHUNK), jnp.int32),
            pltpu.VMEM((2, GATHER_CHUNK, EMBEDDING_DIM), jnp.float32),
            pltpu.SemaphoreType.DMA,
            pltpu.SemaphoreType.DMA,
        ],
    )
    def k(table_hbm, idx_hbm, out_hbm, idx_v, rows_v, g0, g1):
        wid = lax.axis_index("s") * 2 + lax.axis_index("c")
        base_chunk = wid * chunks
        pltpu.sync_copy(idx_hbm.at[wid], idx_v)

        def gather(j, slot, sem):
            pltpu.async_copy(table_hbm.at[idx_v.at[j]], rows_v.at[slot], sem)

        def wait(slot, sem):
            pltpu.make_async_copy(table_hbm.at[idx_v.at[0]], rows_v.at[slot],
                                  sem).wait()

        def write(j, slot):
            pltpu.sync_copy(
                rows_v.at[slot],
                out_hbm.at[pl.ds((base_chunk + j) * GATHER_CHUNK,
                                 GATHER_CHUNK)])

        gather(0, 0, g0)

        def body(g, _):
            jb = 2 * g
            gather(jb + 1, 1, g1)
            wait(0, g0)
            write(jb, 0)

            @pl.when(jb + 2 < chunks)
            def _():
                gather(jb + 2, 0, g0)

            wait(1, g1)
            write(jb + 1, 1)
            return 0

        lax.fori_loop(0, chunks // 2, body, 0, unroll=False)

    return k(table, idx2)


def kernel(input, code, codebook):
    batch, hist = input.shape
    n_total = batch * hist
    info = plsc.get_sparse_core_info()
    nw = info.num_cores * info.num_subcores
    n_chunks = n_total // GATHER_CHUNK
    chunks_per_worker = n_chunks // nw

    code4 = code.reshape(NUM_EMBEDDINGS // PACK, PACK * KD)
    cb2 = codebook.reshape(KD, EMBEDDING_DIM)
    table = _build_table(code4, cb2).reshape(NUM_EMBEDDINGS, EMBEDDING_DIM)

    idx2 = input.reshape(nw, chunks_per_worker, GATHER_CHUNK).astype(jnp.int32)
    out = _gather_rows(table, idx2, n_total, chunks_per_worker, nw)
    return out.reshape(batch, hist, EMBEDDING_DIM)


# speedup vs baseline: 2.4387x; 2.4387x over previous
import functools

import jax
import jax.numpy as jnp
from jax import lax
from jax.experimental import pallas as pl
from jax.experimental.pallas import tpu as pltpu
from jax.experimental.pallas import tpu_sc as plsc

NUM_EMBEDDINGS = 100000
EMBEDDING_DIM = 32
NUM_CODEBOOK = 16
NUM_CODEWORD = 32
KD = NUM_CODEBOOK * NUM_CODEWORD

ROW_BLOCK = 2000
TABLE_D = 128


def _transform_body(code_ref, cb_ref, out_ref):
    c = code_ref[...]
    w = NUM_CODEWORD
    m = jnp.max(c, axis=1, keepdims=True)
    e = jnp.exp(c - m)
    s4 = e[:, 0:128] + e[:, 128:256] + e[:, 256:384] + e[:, 384:512]
    s = (s4[:, 0:w] + s4[:, w:2 * w] + s4[:, 2 * w:3 * w]
         + s4[:, 3 * w:4 * w])
    rs = 1.0 / s
    rs128 = jnp.concatenate([rs, rs, rs, rs], axis=1)
    rsfull = jnp.concatenate([rs128] * 4, axis=1)
    p = e * rsfull
    acc = jnp.dot(p, cb_ref[...], preferred_element_type=jnp.float32)
    out_ref[...] = jnp.concatenate(
        [acc, jnp.zeros((acc.shape[0], TABLE_D - EMBEDDING_DIM), acc.dtype)],
        axis=1)


def _build_table(code2, cb2):
    grid = NUM_EMBEDDINGS // ROW_BLOCK
    return pl.pallas_call(
        _transform_body,
        grid=(grid,),
        in_specs=[
            pl.BlockSpec((ROW_BLOCK, KD), lambda i: (i, 0)),
            pl.BlockSpec((KD, EMBEDDING_DIM), lambda i: (0, 0)),
        ],
        out_specs=pl.BlockSpec((ROW_BLOCK, TABLE_D), lambda i: (i, 0)),
        out_shape=jax.ShapeDtypeStruct((NUM_EMBEDDINGS, TABLE_D), jnp.float32),
    )(code2, cb2)


GATHER_CHUNK = 128


def _gather_rows(table, idx2, n_total, n_chunks_per_worker, nw):
    chunks = n_chunks_per_worker
    mesh = plsc.VectorSubcoreMesh(core_axis_name="c", subcore_axis_name="s")

    @functools.partial(
        pl.kernel,
        mesh=mesh,
        out_type=jax.ShapeDtypeStruct((n_total, EMBEDDING_DIM), jnp.float32),
        compiler_params=pltpu.CompilerParams(use_tc_tiling_on_sc=False),
        scratch_types=[
            pltpu.VMEM((chunks, GATHER_CHUNK), jnp.int32),
            pltpu.VMEM((2, GATHER_CHUNK, TABLE_D), jnp.float32),
            pltpu.SemaphoreType.DMA,
            pltpu.SemaphoreType.DMA,
        ],
    )
    def k(table_hbm, idx_hbm, out_hbm, idx_v, rows_v, g0, g1):
        wid = lax.axis_index("s") * 2 + lax.axis_index("c")
        base_chunk = wid * chunks
        pltpu.sync_copy(idx_hbm.at[wid], idx_v)

        def gather(j, slot, sem):
            pltpu.async_copy(table_hbm.at[idx_v.at[j]], rows_v.at[slot], sem)

        def wait(slot, sem):
            pltpu.make_async_copy(table_hbm.at[idx_v.at[0]], rows_v.at[slot],
                                  sem).wait()

        def write(j, slot):
            pltpu.sync_copy(
                rows_v.at[slot].at[:, pl.ds(0, EMBEDDING_DIM)],
                out_hbm.at[pl.ds((base_chunk + j) * GATHER_CHUNK,
                                 GATHER_CHUNK)])

        gather(0, 0, g0)

        def body(g, _):
            jb = 2 * g
            gather(jb + 1, 1, g1)
            wait(0, g0)
            write(jb, 0)

            @pl.when(jb + 2 < chunks)
            def _():
                gather(jb + 2, 0, g0)

            wait(1, g1)
            write(jb + 1, 1)
            return 0

        lax.fori_loop(0, chunks // 2, body, 0, unroll=False)

    return k(table, idx2)


def kernel(input, code, codebook):
    batch, hist = input.shape
    n_total = batch * hist
    info = plsc.get_sparse_core_info()
    nw = info.num_cores * info.num_subcores
    n_chunks = n_total // GATHER_CHUNK
    chunks_per_worker = n_chunks // nw

    code2 = code.reshape(NUM_EMBEDDINGS, KD)
    cb2 = codebook.reshape(KD, EMBEDDING_DIM)
    table = _build_table(code2, cb2)

    idx2 = input.reshape(nw, chunks_per_worker, GATHER_CHUNK).astype(jnp.int32)
    out = _gather_rows(table, idx2, n_total, chunks_per_worker, nw)
    return out.reshape(batch, hist, EMBEDDING_DIM)


# speedup vs baseline: 2.6408x; 1.0829x over previous
import functools

import jax
import jax.numpy as jnp
from jax import lax
from jax.experimental import pallas as pl
from jax.experimental.pallas import tpu as pltpu
from jax.experimental.pallas import tpu_sc as plsc

NUM_EMBEDDINGS = 100000
EMBEDDING_DIM = 32
NUM_CODEBOOK = 16
NUM_CODEWORD = 32
KD = NUM_CODEBOOK * NUM_CODEWORD

ROW_BLOCK = 2000
TABLE_D = 128


def _transform_body(code_ref, cb_ref, out_ref):
    c = code_ref[...]
    w = NUM_CODEWORD
    m = jnp.max(c, axis=1, keepdims=True)
    e = jnp.exp(c - m)
    s4 = e[:, 0:128] + e[:, 128:256] + e[:, 256:384] + e[:, 384:512]
    s = (s4[:, 0:w] + s4[:, w:2 * w] + s4[:, 2 * w:3 * w]
         + s4[:, 3 * w:4 * w])
    rs = 1.0 / s
    rs128 = jnp.concatenate([rs, rs, rs, rs], axis=1)
    rsfull = jnp.concatenate([rs128] * 4, axis=1)
    p = e * rsfull
    acc = jnp.dot(p, cb_ref[...], preferred_element_type=jnp.float32)
    out_ref[...] = jnp.concatenate(
        [acc, jnp.zeros((acc.shape[0], TABLE_D - EMBEDDING_DIM), acc.dtype)],
        axis=1)


def _build_table(code2, cb2):
    grid = NUM_EMBEDDINGS // ROW_BLOCK
    return pl.pallas_call(
        _transform_body,
        grid=(grid,),
        in_specs=[
            pl.BlockSpec((ROW_BLOCK, KD), lambda i: (i, 0)),
            pl.BlockSpec((KD, EMBEDDING_DIM), lambda i: (0, 0)),
        ],
        out_specs=pl.BlockSpec((ROW_BLOCK, TABLE_D), lambda i: (i, 0)),
        out_shape=jax.ShapeDtypeStruct((NUM_EMBEDDINGS, TABLE_D), jnp.float32),
    )(code2, cb2)


GATHER_CHUNK = 128


def _gather_rows(table, idx1, n_total, n_chunks_per_worker, nw):
    chunks = n_chunks_per_worker
    per_worker = chunks * GATHER_CHUNK
    mesh = plsc.VectorSubcoreMesh(core_axis_name="c", subcore_axis_name="s")

    @functools.partial(
        pl.kernel,
        mesh=mesh,
        out_type=jax.ShapeDtypeStruct((n_total, TABLE_D), jnp.float32),
        compiler_params=pltpu.CompilerParams(use_tc_tiling_on_sc=False),
        scratch_types=[
            pltpu.VMEM((per_worker,), jnp.int32),
            pltpu.VMEM((2, GATHER_CHUNK, TABLE_D), jnp.float32),
            pltpu.SemaphoreType.DMA,
            pltpu.SemaphoreType.DMA,
        ],
    )
    def k(table_hbm, idx_hbm, out_hbm, idx_v, rows_v, g0, g1):
        wid = lax.axis_index("s") * 2 + lax.axis_index("c")
        base_chunk = wid * chunks
        pltpu.sync_copy(idx_hbm.at[pl.ds(wid * per_worker, per_worker)], idx_v)

        def gather(j, slot, sem):
            pltpu.async_copy(
                table_hbm.at[idx_v.at[pl.ds(j * GATHER_CHUNK, GATHER_CHUNK)]],
                rows_v.at[slot], sem)

        def wait(slot, sem):
            pltpu.make_async_copy(
                table_hbm.at[idx_v.at[pl.ds(0, GATHER_CHUNK)]],
                rows_v.at[slot], sem).wait()

        def write(j, slot):
            pltpu.sync_copy(
                rows_v.at[slot],
                out_hbm.at[pl.ds((base_chunk + j) * GATHER_CHUNK,
                                 GATHER_CHUNK)])

        gather(0, 0, g0)

        def body(g, _):
            jb = 2 * g
            gather(jb + 1, 1, g1)
            wait(0, g0)
            write(jb, 0)

            @pl.when(jb + 2 < chunks)
            def _():
                gather(jb + 2, 0, g0)

            wait(1, g1)
            write(jb + 1, 1)
            return 0

        lax.fori_loop(0, chunks // 2, body, 0, unroll=False)

    return k(table, idx1)


def kernel(input, code, codebook):
    batch, hist = input.shape
    n_total = batch * hist
    info = plsc.get_sparse_core_info()
    nw = info.num_cores * info.num_subcores
    n_chunks = n_total // GATHER_CHUNK
    chunks_per_worker = n_chunks // nw

    code2 = code.reshape(NUM_EMBEDDINGS, KD)
    cb2 = codebook.reshape(KD, EMBEDDING_DIM)
    table = _build_table(code2, cb2)

    idx1 = input.reshape(n_total).astype(jnp.int32)
    out = _gather_rows(table, idx1, n_total, chunks_per_worker, nw)
    return out[:, :EMBEDDING_DIM].reshape(batch, hist, EMBEDDING_DIM)


# speedup vs baseline: 3.2170x; 1.2182x over previous
import functools

import jax
import jax.numpy as jnp
from jax import lax
from jax.experimental import pallas as pl
from jax.experimental.pallas import tpu as pltpu
from jax.experimental.pallas import tpu_sc as plsc

NUM_EMBEDDINGS = 100000
EMBEDDING_DIM = 32
NUM_CODEBOOK = 16
NUM_CODEWORD = 32
KD = NUM_CODEBOOK * NUM_CODEWORD

ROW_BLOCK = 2000
TABLE_D = 128


def _transform_body(code_ref, cb_ref, out_ref):
    c = code_ref[...]
    w = NUM_CODEWORD
    m = jnp.max(c, axis=1, keepdims=True)
    e = jnp.exp(c - m)
    s4 = e[:, 0:128] + e[:, 128:256] + e[:, 256:384] + e[:, 384:512]
    s = (s4[:, 0:w] + s4[:, w:2 * w] + s4[:, 2 * w:3 * w]
         + s4[:, 3 * w:4 * w])
    rs = 1.0 / s
    rs128 = jnp.concatenate([rs, rs, rs, rs], axis=1)
    rsfull = jnp.concatenate([rs128] * 4, axis=1)
    p = e * rsfull
    acc = jnp.dot(p, cb_ref[...], preferred_element_type=jnp.float32)
    out_ref[...] = jnp.concatenate(
        [acc, jnp.zeros((acc.shape[0], TABLE_D - EMBEDDING_DIM), acc.dtype)],
        axis=1)


def _build_table(code2, cb2):
    grid = NUM_EMBEDDINGS // ROW_BLOCK
    return pl.pallas_call(
        _transform_body,
        grid=(grid,),
        in_specs=[
            pl.BlockSpec((ROW_BLOCK, KD), lambda i: (i, 0)),
            pl.BlockSpec((KD, EMBEDDING_DIM), lambda i: (0, 0)),
        ],
        out_specs=pl.BlockSpec((ROW_BLOCK, TABLE_D), lambda i: (i, 0)),
        out_shape=jax.ShapeDtypeStruct((NUM_EMBEDDINGS, TABLE_D), jnp.float32),
    )(code2, cb2)


BPC = 2


def _gather_rows(table, idx2d, batch, hist, nw):
    chunk_idx = BPC * hist
    bw = batch // nw
    chunks = bw // BPC
    mesh = plsc.VectorSubcoreMesh(core_axis_name="c", subcore_axis_name="s")

    @functools.partial(
        pl.kernel,
        mesh=mesh,
        out_type=jax.ShapeDtypeStruct((batch, hist, EMBEDDING_DIM),
                                      jnp.float32),
        scratch_types=[
            pltpu.VMEM((chunks, chunk_idx), jnp.int32),
            pltpu.VMEM((2, chunk_idx, TABLE_D), jnp.float32),
            pltpu.VMEM((2, BPC, hist, EMBEDDING_DIM), jnp.float32),
            pltpu.SemaphoreType.DMA,
            pltpu.SemaphoreType.DMA,
        ],
    )
    def k(table_hbm, idx_hbm, out_hbm, idx_v, rows_v, small_v, g0, g1):
        wid = lax.axis_index("s") * 2 + lax.axis_index("c")
        base_b = wid * bw
        pltpu.sync_copy(idx_hbm.at[pl.ds(wid * chunks, chunks)], idx_v)

        def gather(j, slot, sem):
            pltpu.async_copy(table_hbm.at[idx_v.at[j]], rows_v.at[slot], sem)

        def wait(slot, sem):
            pltpu.make_async_copy(table_hbm.at[idx_v.at[0]], rows_v.at[slot],
                                  sem).wait()

        def drain(j, slot):
            big = rows_v.at[slot]
            sm = small_v.at[slot]

            def row(h, _):
                for q in range(BPC):
                    src = big.at[q * hist + h]
                    dst = sm.at[q].at[h]
                    dst[pl.ds(0, 16)] = src[pl.ds(0, 16)]
                    dst[pl.ds(16, 16)] = src[pl.ds(16, 16)]
                return 0

            lax.fori_loop(0, hist, row, 0, unroll=2)
            pltpu.sync_copy(sm, out_hbm.at[pl.ds(base_b + j * BPC, BPC)])

        gather(0, 0, g0)

        def body(g, _):
            jb = 2 * g
            gather(jb + 1, 1, g1)
            wait(0, g0)
            drain(jb, 0)

            @pl.when(jb + 2 < chunks)
            def _():
                gather(jb + 2, 0, g0)

            wait(1, g1)
            drain(jb + 1, 1)
            return 0

        lax.fori_loop(0, chunks // 2, body, 0, unroll=False)

    return k(table, idx2d)


def kernel(input, code, codebook):
    batch, hist = input.shape
    info = plsc.get_sparse_core_info()
    nw = info.num_cores * info.num_subcores

    code2 = code.reshape(NUM_EMBEDDINGS, KD)
    cb2 = codebook.reshape(KD, EMBEDDING_DIM)
    table = _build_table(code2, cb2)

    idx2d = input.reshape(batch // BPC, BPC * hist).astype(jnp.int32)
    return _gather_rows(table, idx2d, batch, hist, nw)


# speedup vs baseline: 3.2771x; 1.0187x over previous
import functools

import jax
import jax.numpy as jnp
from jax import lax
from jax.experimental import pallas as pl
from jax.experimental.pallas import tpu as pltpu
from jax.experimental.pallas import tpu_sc as plsc

NUM_EMBEDDINGS = 100000
EMBEDDING_DIM = 32
NUM_CODEBOOK = 16
NUM_CODEWORD = 32
KD = NUM_CODEBOOK * NUM_CODEWORD

ROW_BLOCK = 2000
TABLE_D = 128


def _transform_body(code_ref, cb_ref, out_ref):
    c = code_ref[...]
    w = NUM_CODEWORD
    m = jnp.max(c, axis=1, keepdims=True)
    e = jnp.exp(c - m)
    s4 = e[:, 0:128] + e[:, 128:256] + e[:, 256:384] + e[:, 384:512]
    s = (s4[:, 0:w] + s4[:, w:2 * w] + s4[:, 2 * w:3 * w]
         + s4[:, 3 * w:4 * w])
    rs = 1.0 / s
    rs128 = jnp.concatenate([rs, rs, rs, rs], axis=1)
    rsfull = jnp.concatenate([rs128] * 4, axis=1)
    p = e * rsfull
    acc = jnp.dot(p, cb_ref[...], preferred_element_type=jnp.float32)
    out_ref[...] = jnp.concatenate(
        [acc, jnp.zeros((acc.shape[0], TABLE_D - EMBEDDING_DIM), acc.dtype)],
        axis=1)


def _build_table(code2, cb2):
    grid = NUM_EMBEDDINGS // ROW_BLOCK
    return pl.pallas_call(
        _transform_body,
        grid=(grid,),
        in_specs=[
            pl.BlockSpec((ROW_BLOCK, KD), lambda i: (i, 0)),
            pl.BlockSpec((KD, EMBEDDING_DIM), lambda i: (0, 0)),
        ],
        out_specs=pl.BlockSpec((ROW_BLOCK, TABLE_D), lambda i: (i, 0)),
        out_shape=jax.ShapeDtypeStruct((NUM_EMBEDDINGS, TABLE_D), jnp.float32),
    )(code2, cb2)


SUP = 4
HALF = 2


def _gather_rows(table, idx1, batch, hist, nw):
    sup_idx = SUP * hist
    bw = batch // nw
    sups = bw // SUP
    per_worker = bw * hist
    g_hi = (sup_idx // 128) * 128
    g_lo = sup_idx - g_hi
    mesh = plsc.VectorSubcoreMesh(core_axis_name="c", subcore_axis_name="s")

    @functools.partial(
        pl.kernel,
        mesh=mesh,
        out_type=jax.ShapeDtypeStruct((batch, hist, EMBEDDING_DIM),
                                      jnp.float32),
        scratch_types=[
            pltpu.VMEM((per_worker,), jnp.int32),
            pltpu.VMEM((2, sup_idx, TABLE_D), jnp.float32),
            pltpu.VMEM((2, HALF, hist, EMBEDDING_DIM), jnp.float32),
            pltpu.SemaphoreType.DMA,
            pltpu.SemaphoreType.DMA,
        ],
    )
    def k(table_hbm, idx_hbm, out_hbm, idx_v, rows_v, small_v, g0, g1):
        wid = lax.axis_index("s") * 2 + lax.axis_index("c")
        base_b = wid * bw
        pltpu.sync_copy(idx_hbm.at[pl.ds(wid * per_worker, per_worker)],
                        idx_v)

        def gather(s, slot, sem):
            pltpu.async_copy(
                table_hbm.at[idx_v.at[pl.ds(s * sup_idx, g_hi)]],
                rows_v.at[slot].at[pl.ds(0, g_hi)], sem)
            pltpu.async_copy(
                table_hbm.at[idx_v.at[pl.ds(s * sup_idx + g_hi, g_lo)]],
                rows_v.at[slot].at[pl.ds(g_hi, g_lo)], sem)

        def wait(slot, sem):
            pltpu.make_async_copy(
                table_hbm.at[idx_v.at[pl.ds(0, g_hi)]],
                rows_v.at[slot].at[pl.ds(0, g_hi)], sem).wait()
            pltpu.make_async_copy(
                table_hbm.at[idx_v.at[pl.ds(0, g_lo)]],
                rows_v.at[slot].at[pl.ds(g_hi, g_lo)], sem).wait()

        def drain(s, slot):
            big = rows_v.at[slot]
            for half in range(SUP // HALF):
                sm = small_v.at[slot]

                def row(h, _):
                    for t in range(HALF):
                        src = big.at[(half * HALF + t) * hist + h]
                        dst = sm.at[t].at[h]
                        dst[pl.ds(0, 16)] = src[pl.ds(0, 16)]
                        dst[pl.ds(16, 16)] = src[pl.ds(16, 16)]
                    return 0

                lax.fori_loop(0, hist, row, 0, unroll=2)
                pltpu.sync_copy(
                    sm,
                    out_hbm.at[pl.ds(base_b + s * SUP + half * HALF, HALF)])

        gather(0, 0, g0)

        def body(g, _):
            sb = 2 * g
            gather(sb + 1, 1, g1)
            wait(0, g0)
            drain(sb, 0)

            @pl.when(sb + 2 < sups)
            def _():
                gather(sb + 2, 0, g0)

            wait(1, g1)
            drain(sb + 1, 1)
            return 0

        lax.fori_loop(0, sups // 2, body, 0, unroll=False)

    return k(table, idx1)


def kernel(input, code, codebook):
    batch, hist = input.shape
    info = plsc.get_sparse_core_info()
    nw = info.num_cores * info.num_subcores

    code2 = code.reshape(NUM_EMBEDDINGS, KD)
    cb2 = codebook.reshape(KD, EMBEDDING_DIM)
    table = _build_table(code2, cb2)

    idx1 = input.reshape(batch * hist).astype(jnp.int32)
    return _gather_rows(table, idx1, batch, hist, nw)


# speedup vs baseline: 3.2919x; 1.0045x over previous
import functools

import jax
import jax.numpy as jnp
from jax import lax
from jax.experimental import pallas as pl
from jax.experimental.pallas import tpu as pltpu
from jax.experimental.pallas import tpu_sc as plsc

NUM_EMBEDDINGS = 100000
EMBEDDING_DIM = 32
NUM_CODEBOOK = 16
NUM_CODEWORD = 32
KD = NUM_CODEBOOK * NUM_CODEWORD

ROW_BLOCK = 2000
TABLE_D = 128


def _transform_body(code_ref, cb_ref, out_ref):
    c = code_ref[...]
    w = NUM_CODEWORD
    m = jnp.max(c, axis=1, keepdims=True)
    e = jnp.exp(c - m)
    s4 = e[:, 0:128] + e[:, 128:256] + e[:, 256:384] + e[:, 384:512]
    s = (s4[:, 0:w] + s4[:, w:2 * w] + s4[:, 2 * w:3 * w]
         + s4[:, 3 * w:4 * w])
    rs = 1.0 / s
    rs128 = jnp.concatenate([rs, rs, rs, rs], axis=1)
    rsfull = jnp.concatenate([rs128] * 4, axis=1)
    p = e * rsfull
    acc = jnp.dot(p, cb_ref[...], preferred_element_type=jnp.float32)
    out_ref[...] = jnp.concatenate(
        [acc, jnp.zeros((acc.shape[0], TABLE_D - EMBEDDING_DIM), acc.dtype)],
        axis=1)


def _build_table(code2, cb2):
    grid = NUM_EMBEDDINGS // ROW_BLOCK
    return pl.pallas_call(
        _transform_body,
        grid=(grid,),
        in_specs=[
            pl.BlockSpec((ROW_BLOCK, KD), lambda i: (i, 0)),
            pl.BlockSpec((KD, EMBEDDING_DIM), lambda i: (0, 0)),
        ],
        out_specs=pl.BlockSpec((ROW_BLOCK, TABLE_D), lambda i: (i, 0)),
        out_shape=jax.ShapeDtypeStruct((NUM_EMBEDDINGS, TABLE_D), jnp.float32),
    )(code2, cb2)


RING = 4


def _gather_rows(table, idx, batch, hist, nw):
    bw = batch // nw
    mesh = plsc.VectorSubcoreMesh(core_axis_name="c", subcore_axis_name="s")

    @functools.partial(
        pl.kernel,
        mesh=mesh,
        out_type=jax.ShapeDtypeStruct((batch, hist, EMBEDDING_DIM),
                                      jnp.float32),
        scratch_types=[
            pltpu.VMEM((bw, hist), jnp.int32),
            pltpu.VMEM((RING, hist, TABLE_D), jnp.float32),
            pltpu.VMEM((RING, hist, EMBEDDING_DIM), jnp.float32),
        ] + [pltpu.SemaphoreType.DMA] * (2 * RING),
    )
    def k(table_hbm, idx_hbm, out_hbm, idx_v, rows_v, small_v, *sems):
        gsem = sems[:RING]
        wsem = sems[RING:]
        wid = lax.axis_index("s") * 2 + lax.axis_index("c")
        base_b = wid * bw
        pltpu.sync_copy(idx_hbm.at[pl.ds(base_b, bw)], idx_v)

        def gather(b, r):
            pltpu.async_copy(table_hbm.at[idx_v.at[b]], rows_v.at[r],
                             gsem[r])

        def gwait(r):
            pltpu.make_async_copy(table_hbm.at[idx_v.at[0]], rows_v.at[r],
                                  gsem[r]).wait()

        def wwait(r):
            pltpu.make_async_copy(small_v.at[r], out_hbm.at[0],
                                  wsem[r]).wait()

        def extract(r):
            big = rows_v.at[r]
            sm = small_v.at[r]

            def row(h, _):
                src = big.at[h]
                dst = sm.at[h]
                dst[pl.ds(0, 16)] = src[pl.ds(0, 16)]
                dst[pl.ds(16, 16)] = src[pl.ds(16, 16)]
                return 0

            lax.fori_loop(0, hist, row, 0, unroll=2)

        for r in range(RING - 1):
            gather(r, r)

        def body(g, _):
            for r in range(RING):
                b = RING * g + r
                gwait(r)

                @pl.when(b >= RING)
                def _():
                    wwait(r)

                extract(r)
                pltpu.async_copy(small_v.at[r], out_hbm.at[base_b + b],
                                 wsem[r])

                @pl.when(b + RING - 1 < bw)
                def _():
                    gather(b + RING - 1, (r + RING - 1) % RING)

            return 0

        lax.fori_loop(0, bw // RING, body, 0, unroll=False)
        for r in range(RING):
            wwait(r)

    return k(table, idx)


def kernel(input, code, codebook):
    batch, hist = input.shape
    info = plsc.get_sparse_core_info()
    nw = info.num_cores * info.num_subcores

    code2 = code.reshape(NUM_EMBEDDINGS, KD)
    cb2 = codebook.reshape(KD, EMBEDDING_DIM)
    table = _build_table(code2, cb2)

    return _gather_rows(table, input.astype(jnp.int32), batch, hist, nw)


# speedup vs baseline: 3.4004x; 1.0330x over previous
import functools

import jax
import jax.numpy as jnp
from jax import lax
from jax.experimental import pallas as pl
from jax.experimental.pallas import tpu as pltpu
from jax.experimental.pallas import tpu_sc as plsc

NUM_EMBEDDINGS = 100000
EMBEDDING_DIM = 32
NUM_CODEBOOK = 16
NUM_CODEWORD = 32
KD = NUM_CODEBOOK * NUM_CODEWORD

ROW_BLOCK = 4000
TABLE_D = 128


def _transform_body(code_ref, cb_ref, out_ref):
    c = code_ref[...]
    w = NUM_CODEWORD
    m = jnp.max(c, axis=1, keepdims=True)
    e = jnp.exp(c - m)
    s4 = e[:, 0:128] + e[:, 128:256] + e[:, 256:384] + e[:, 384:512]
    s = (s4[:, 0:w] + s4[:, w:2 * w] + s4[:, 2 * w:3 * w]
         + s4[:, 3 * w:4 * w])
    rs = 1.0 / s
    rs128 = jnp.concatenate([rs, rs, rs, rs], axis=1)
    rsfull = jnp.concatenate([rs128] * 4, axis=1)
    p = e * rsfull
    acc = jnp.dot(p, cb_ref[...], preferred_element_type=jnp.float32)
    out_ref[...] = jnp.concatenate(
        [acc, jnp.zeros((acc.shape[0], TABLE_D - EMBEDDING_DIM), acc.dtype)],
        axis=1)


def _build_table(code2, cb2):
    grid = NUM_EMBEDDINGS // ROW_BLOCK
    return pl.pallas_call(
        _transform_body,
        grid=(grid,),
        in_specs=[
            pl.BlockSpec((ROW_BLOCK, KD), lambda i: (i, 0)),
            pl.BlockSpec((KD, EMBEDDING_DIM), lambda i: (0, 0)),
        ],
        out_specs=pl.BlockSpec((ROW_BLOCK, TABLE_D), lambda i: (i, 0)),
        out_shape=jax.ShapeDtypeStruct((NUM_EMBEDDINGS, TABLE_D), jnp.float32),
    )(code2, cb2)


RING = 4


def _gather_rows(table, idx, batch, hist, nw):
    bw = batch // nw
    mesh = plsc.VectorSubcoreMesh(core_axis_name="c", subcore_axis_name="s")

    @functools.partial(
        pl.kernel,
        mesh=mesh,
        out_type=jax.ShapeDtypeStruct((batch, hist, EMBEDDING_DIM),
                                      jnp.float32),
        scratch_types=[
            pltpu.VMEM((bw, hist), jnp.int32),
            pltpu.VMEM((RING, hist, TABLE_D), jnp.float32),
            pltpu.VMEM((RING, hist, EMBEDDING_DIM), jnp.float32),
        ] + [pltpu.SemaphoreType.DMA] * (2 * RING),
    )
    def k(table_hbm, idx_hbm, out_hbm, idx_v, rows_v, small_v, *sems):
        gsem = sems[:RING]
        wsem = sems[RING:]
        wid = lax.axis_index("s") * 2 + lax.axis_index("c")
        base_b = wid * bw
        pltpu.sync_copy(idx_hbm.at[pl.ds(base_b, bw)], idx_v)

        def gather(b, r):
            pltpu.async_copy(table_hbm.at[idx_v.at[b]], rows_v.at[r],
                             gsem[r])

        def gwait(r):
            pltpu.make_async_copy(table_hbm.at[idx_v.at[0]], rows_v.at[r],
                                  gsem[r]).wait()

        def wwait(r):
            pltpu.make_async_copy(small_v.at[r], out_hbm.at[0],
                                  wsem[r]).wait()

        def extract(r):
            big = rows_v.at[r]
            sm = small_v.at[r]

            def row(h, _):
                src = big.at[h]
                dst = sm.at[h]
                dst[pl.ds(0, 16)] = src[pl.ds(0, 16)]
                dst[pl.ds(16, 16)] = src[pl.ds(16, 16)]
                return 0

            lax.fori_loop(0, hist, row, 0, unroll=2)

        for r in range(RING - 1):
            gather(r, r)

        def body(g, _):
            for r in range(RING):
                b = RING * g + r
                gwait(r)

                @pl.when(b >= RING)
                def _():
                    wwait(r)

                extract(r)
                pltpu.async_copy(small_v.at[r], out_hbm.at[base_b + b],
                                 wsem[r])

                @pl.when(b + RING - 1 < bw)
                def _():
                    gather(b + RING - 1, (r + RING - 1) % RING)

            return 0

        lax.fori_loop(0, bw // RING, body, 0, unroll=False)
        for r in range(RING):
            wwait(r)

    return k(table, idx)


def kernel(input, code, codebook):
    batch, hist = input.shape
    info = plsc.get_sparse_core_info()
    nw = info.num_cores * info.num_subcores

    code2 = code.reshape(NUM_EMBEDDINGS, KD)
    cb2 = codebook.reshape(KD, EMBEDDING_DIM)
    table = _build_table(code2, cb2)

    return _gather_rows(table, input.astype(jnp.int32), batch, hist, nw)


# speedup vs baseline: 3.4593x; 1.0173x over previous
import functools

import jax
import jax.numpy as jnp
from jax import lax
from jax.experimental import pallas as pl
from jax.experimental.pallas import tpu as pltpu
from jax.experimental.pallas import tpu_sc as plsc

NUM_EMBEDDINGS = 100000
EMBEDDING_DIM = 32
NUM_CODEBOOK = 16
NUM_CODEWORD = 32
KD = NUM_CODEBOOK * NUM_CODEWORD

ROW_BLOCK = 5000
TABLE_D = 128


def _transform_body(code_ref, cb_ref, out_ref):
    c = code_ref[...]
    w = NUM_CODEWORD
    m = jnp.max(c, axis=1, keepdims=True)
    e = jnp.exp(c - m)
    s4 = e[:, 0:128] + e[:, 128:256] + e[:, 256:384] + e[:, 384:512]
    s = (s4[:, 0:w] + s4[:, w:2 * w] + s4[:, 2 * w:3 * w]
         + s4[:, 3 * w:4 * w])
    rs = 1.0 / s
    rs128 = jnp.concatenate([rs, rs, rs, rs], axis=1)
    rsfull = jnp.concatenate([rs128] * 4, axis=1)
    p = e * rsfull
    acc = jnp.dot(p, cb_ref[...], preferred_element_type=jnp.float32)
    out_ref[...] = jnp.concatenate(
        [acc, jnp.zeros((acc.shape[0], TABLE_D - EMBEDDING_DIM), acc.dtype)],
        axis=1)


def _build_table(code2, cb2):
    grid = NUM_EMBEDDINGS // ROW_BLOCK
    return pl.pallas_call(
        _transform_body,
        grid=(grid,),
        in_specs=[
            pl.BlockSpec((ROW_BLOCK, KD), lambda i: (i, 0)),
            pl.BlockSpec((KD, EMBEDDING_DIM), lambda i: (0, 0)),
        ],
        out_specs=pl.BlockSpec((ROW_BLOCK, TABLE_D), lambda i: (i, 0)),
        out_shape=jax.ShapeDtypeStruct((NUM_EMBEDDINGS, TABLE_D), jnp.float32),
    )(code2, cb2)


RING = 8


def _gather_rows(table, idx, batch, hist, nw):
    bw = batch // nw
    mesh = plsc.VectorSubcoreMesh(core_axis_name="c", subcore_axis_name="s")

    @functools.partial(
        pl.kernel,
        mesh=mesh,
        out_type=jax.ShapeDtypeStruct((batch, hist, EMBEDDING_DIM),
                                      jnp.float32),
        scratch_types=[
            pltpu.VMEM((bw, hist), jnp.int32),
            pltpu.VMEM((RING, hist, TABLE_D), jnp.float32),
            pltpu.VMEM((RING, hist, EMBEDDING_DIM), jnp.float32),
        ] + [pltpu.SemaphoreType.DMA] * (2 * RING),
    )
    def k(table_hbm, idx_hbm, out_hbm, idx_v, rows_v, small_v, *sems):
        gsem = sems[:RING]
        wsem = sems[RING:]
        wid = lax.axis_index("s") * 2 + lax.axis_index("c")
        base_b = wid * bw
        pltpu.sync_copy(idx_hbm.at[pl.ds(base_b, bw)], idx_v)

        def gather(b, r):
            pltpu.async_copy(table_hbm.at[idx_v.at[b]], rows_v.at[r],
                             gsem[r])

        def gwait(r):
            pltpu.make_async_copy(table_hbm.at[idx_v.at[0]], rows_v.at[r],
                                  gsem[r]).wait()

        def wwait(r):
            pltpu.make_async_copy(small_v.at[r], out_hbm.at[0],
                                  wsem[r]).wait()

        def extract(r):
            big = rows_v.at[r]
            sm = small_v.at[r]

            def row(h, _):
                src = big.at[h]
                dst = sm.at[h]
                dst[pl.ds(0, 16)] = src[pl.ds(0, 16)]
                dst[pl.ds(16, 16)] = src[pl.ds(16, 16)]
                return 0

            lax.fori_loop(0, hist, row, 0, unroll=2)

        for r in range(RING - 1):
            gather(r, r)

        def body(g, _):
            for r in range(RING):
                b = RING * g + r
                gwait(r)

                @pl.when(b >= RING)
                def _():
                    wwait(r)

                extract(r)
                pltpu.async_copy(small_v.at[r], out_hbm.at[base_b + b],
                                 wsem[r])

                @pl.when(b + RING - 1 < bw)
                def _():
                    gather(b + RING - 1, (r + RING - 1) % RING)

            return 0

        lax.fori_loop(0, bw // RING, body, 0, unroll=False)
        for r in range(RING):
            wwait(r)

    return k(table, idx)


def kernel(input, code, codebook):
    batch, hist = input.shape
    info = plsc.get_sparse_core_info()
    nw = info.num_cores * info.num_subcores

    code2 = code.reshape(NUM_EMBEDDINGS, KD)
    cb2 = codebook.reshape(KD, EMBEDDING_DIM)
    table = _build_table(code2, cb2)

    return _gather_rows(table, input.astype(jnp.int32), batch, hist, nw)
